# Initial kernel scaffold; baseline (speedup 1.0000x reference)
import jax
import jax.numpy as jnp
from jax.experimental import pallas as pl


def kernel(x):
    raise NotImplementedError("write your pallas kernel here")


# speedup vs baseline: 119.3186x; 119.3186x over previous
import functools

import jax
import jax.numpy as jnp
from jax import lax
from jax.experimental import pallas as pl
from jax.experimental.pallas import tpu as pltpu
from jax.experimental.pallas import tpu_sc as plsc

Z, H, W = 14, 32, 64
C = 70
POS = Z * H * W
NC, NS, LANES = 2, 16, 16
NW = NC * NS
COLS = POS // NW
CHUNKS = COLS // LANES


def _fold_body(x_hbm, out_hbm, buf, cnt_v, rcp_v):
    cid = lax.axis_index("c")
    sid = lax.axis_index("s")
    wid = sid * NC + cid
    base = wid * COLS

    pltpu.sync_copy(x_hbm.at[:, pl.ds(base, COLS)], buf)

    def count_iter(j, carry):
        p = base + j * LANES + lax.broadcasted_iota(jnp.int32, (LANES,), 0)
        z = p >> 11
        h = (p >> 6) & (H - 1)
        cz = jnp.minimum(z + 1, Z - 2) - jnp.maximum(z - 1, 1) + 1
        ch = jnp.minimum(h + 1, H - 2) - jnp.maximum(h - 1, 1) + 1
        cnt = (cz * ch * 3).astype(jnp.float32)
        cnt_v[pl.ds(j * LANES, LANES)] = cnt
        rcp_v[pl.ds(j * LANES, LANES)] = 1.0 / cnt
        return carry

    lax.fori_loop(0, CHUNKS, count_iter, 0)

    def chan_iter(c, carry):
        def col_iter(j, inner):
            sl = pl.ds(j * LANES, LANES)
            v = buf[c, sl]
            acc = v * cnt_v[sl]
            buf[c, sl] = acc * rcp_v[sl]
            return inner
        return lax.fori_loop(0, CHUNKS, col_iter, carry)

    lax.fori_loop(0, C, chan_iter, 0)

    pltpu.sync_copy(buf, out_hbm.at[:, pl.ds(base, COLS)])


@functools.partial(
    pl.kernel,
    mesh=plsc.VectorSubcoreMesh(core_axis_name="c", subcore_axis_name="s"),
    out_type=jax.ShapeDtypeStruct((C, POS), jnp.float32),
    scratch_types=[
        pltpu.VMEM((C, COLS), jnp.float32),
        pltpu.VMEM((COLS,), jnp.float32),
        pltpu.VMEM((COLS,), jnp.float32),
    ],
)
def _fold_sc(x_hbm, out_hbm, buf, cnt_v, rcp_v):
    _fold_body(x_hbm, out_hbm, buf, cnt_v, rcp_v)


def kernel(x):
    x2 = x.reshape(C, POS)
    y = _fold_sc(x2)
    return y.reshape(1, C, Z, H, W)


# speedup vs baseline: 169.5037x; 1.4206x over previous
import functools

import jax
import jax.numpy as jnp
from jax import lax
from jax.experimental import pallas as pl
from jax.experimental.pallas import tpu as pltpu
from jax.experimental.pallas import tpu_sc as plsc

Z, H, W = 14, 32, 64
C = 70
POS = Z * H * W
NC, NS, LANES = 2, 16, 16
NW = NC * NS
COLS = POS // NW
CHUNKS = COLS // LANES


def _fold_body(x_hbm, out_hbm, buf):
    cid = lax.axis_index("c")
    sid = lax.axis_index("s")
    wid = sid * NC + cid
    base = wid * COLS

    pltpu.sync_copy(x_hbm.at[:, pl.ds(base, COLS)], buf)

    def chunk_iter(j, carry):
        p = base + j * LANES + lax.broadcasted_iota(jnp.int32, (LANES,), 0)
        z = p >> 11
        h = (p >> 6) & (H - 1)
        cz = jnp.minimum(z + 1, Z - 2) - jnp.maximum(z - 1, 1) + 1
        ch = jnp.minimum(h + 1, H - 2) - jnp.maximum(h - 1, 1) + 1
        cnt = (cz * ch * 3).astype(jnp.float32)
        rcp = 1.0 / cnt
        sl = pl.ds(j * LANES, LANES)
        for c in range(C):
            acc = buf[c, sl] * cnt
            buf[c, sl] = acc * rcp
        return carry

    lax.fori_loop(0, CHUNKS, chunk_iter, 0)

    pltpu.sync_copy(buf, out_hbm.at[:, pl.ds(base, COLS)])


@functools.partial(
    pl.kernel,
    mesh=plsc.VectorSubcoreMesh(core_axis_name="c", subcore_axis_name="s"),
    out_type=jax.ShapeDtypeStruct((C, POS), jnp.float32),
    scratch_types=[
        pltpu.VMEM((C, COLS), jnp.float32),
    ],
)
def _fold_sc(x_hbm, out_hbm, buf):
    _fold_body(x_hbm, out_hbm, buf)


def kernel(x):
    x2 = x.reshape(C, POS)
    y = _fold_sc(x2)
    return y.reshape(1, C, Z, H, W)


# speedup vs baseline: 266.3710x; 1.5715x over previous
import functools

import jax
import jax.numpy as jnp
from jax import lax
from jax.experimental import pallas as pl
from jax.experimental.pallas import tpu as pltpu
from jax.experimental.pallas import tpu_sc as plsc

Z, H, W = 14, 32, 64
C = 70
NC, NS, LANES = 2, 16, 16
CSTARTS = (0, 16, 32, 48, C - LANES)


def _fold_body(x_hbm, out_hbm, buf):
    cid = lax.axis_index("c")
    sid = lax.axis_index("s")
    h = sid * NC + cid

    pltpu.sync_copy(x_hbm.at[0, :, h, :, :], buf)

    ch = jnp.minimum(h + 1, H - 2) - jnp.maximum(h - 1, 1) + 1

    def z_iter(z, carry):
        cz = jnp.minimum(z + 1, Z - 2) - jnp.maximum(z - 1, 1) + 1
        cnt = jnp.full((LANES,), (cz * ch * 3).astype(jnp.float32))
        rcp = 1.0 / cnt
        for w in range(W):
            for c0 in CSTARTS:
                sl = pl.ds(c0, LANES)
                acc = buf[z, w, sl] * cnt
                buf[z, w, sl] = acc * rcp
        return carry

    lax.fori_loop(0, Z, z_iter, 0)

    pltpu.sync_copy(buf, out_hbm.at[0, :, h, :, :])


@functools.partial(
    pl.kernel,
    mesh=plsc.VectorSubcoreMesh(core_axis_name="c", subcore_axis_name="s"),
    out_type=jax.ShapeDtypeStruct((1, Z, H, W, C), jnp.float32),
    scratch_types=[
        pltpu.VMEM((Z, W, C), jnp.float32),
    ],
)
def _fold_sc(x_hbm, out_hbm, buf):
    _fold_body(x_hbm, out_hbm, buf)


def kernel(x):
    xt = jnp.transpose(x, (0, 2, 3, 4, 1))
    yt = _fold_sc(xt)
    return jnp.transpose(yt, (0, 4, 1, 2, 3))


# speedup vs baseline: 268.6048x; 1.0084x over previous
import functools

import jax
import jax.numpy as jnp
from jax import lax
from jax.experimental import pallas as pl
from jax.experimental.pallas import tpu as pltpu
from jax.experimental.pallas import tpu_sc as plsc

Z, H, W = 14, 32, 64
C = 70
NC, NS, LANES = 2, 16, 16
CSTARTS = (0, 16, 32, 48, C - LANES)


ZH = Z // 2


def _fold_body(x_hbm, out_hbm, buf_a, buf_b, sem_a, sem_b):
    cid = lax.axis_index("c")
    sid = lax.axis_index("s")
    h = sid * NC + cid

    in_a = pltpu.async_copy(x_hbm.at[0, pl.ds(0, ZH), h, :, :], buf_a, sem_a)
    in_b = pltpu.async_copy(x_hbm.at[0, pl.ds(ZH, ZH), h, :, :], buf_b, sem_b)

    ch = jnp.minimum(h + 1, H - 2) - jnp.maximum(h - 1, 1) + 1

    def make_z_iter(buf, zoff):
        def z_iter(zi, carry):
            z = zi + zoff
            cz = jnp.minimum(z + 1, Z - 2) - jnp.maximum(z - 1, 1) + 1
            cnt = jnp.full((LANES,), (cz * ch * 3).astype(jnp.float32))
            rcp = 1.0 / cnt
            for w in range(W):
                for c0 in CSTARTS:
                    sl = pl.ds(c0, LANES)
                    acc = buf[zi, w, sl] * cnt
                    buf[zi, w, sl] = acc * rcp
            return carry
        return z_iter

    in_a.wait()
    lax.fori_loop(0, ZH, make_z_iter(buf_a, 0), 0)
    out_a = pltpu.async_copy(buf_a, out_hbm.at[0, pl.ds(0, ZH), h, :, :], sem_a)

    in_b.wait()
    lax.fori_loop(0, ZH, make_z_iter(buf_b, ZH), 0)
    out_b = pltpu.async_copy(buf_b, out_hbm.at[0, pl.ds(ZH, ZH), h, :, :], sem_b)

    out_a.wait()
    out_b.wait()


@functools.partial(
    pl.kernel,
    mesh=plsc.VectorSubcoreMesh(core_axis_name="c", subcore_axis_name="s"),
    out_type=jax.ShapeDtypeStruct((1, Z, H, W, C), jnp.float32),
    scratch_types=[
        pltpu.VMEM((ZH, W, C), jnp.float32),
        pltpu.VMEM((ZH, W, C), jnp.float32),
        pltpu.SemaphoreType.DMA,
        pltpu.SemaphoreType.DMA,
    ],
)
def _fold_sc(x_hbm, out_hbm, buf_a, buf_b, sem_a, sem_b):
    _fold_body(x_hbm, out_hbm, buf_a, buf_b, sem_a, sem_b)


def kernel(x):
    xt = jnp.transpose(x, (0, 2, 3, 4, 1))
    yt = _fold_sc(xt)
    return jnp.transpose(yt, (0, 4, 1, 2, 3))
